# baseline (speedup 1.0000x reference)
import functools

import jax
import jax.numpy as jnp
from jax import lax
from jax.experimental import pallas as pl
from jax.experimental.pallas import tpu as pltpu
from jax.experimental.pallas import tpu_sc as plsc

NC = 2
NS = 16
LN = 16

H = 4
F = 32
HF = H * F
CHUNK = 128
DENW = 16


def _prep_body(feat_ref, wt_ref, al_ref, ar_ref, h_ref, elp_ref, erp_ref):
    hb = jnp.dot(feat_ref[...], wt_ref[...], preferred_element_type=jnp.float32)
    h_ref[...] = hb
    elp_ref[...] = jnp.dot(hb, al_ref[...], preferred_element_type=jnp.float32)
    erp_ref[...] = jnp.dot(hb, ar_ref[...], preferred_element_type=jnp.float32)


def _finish_body(part_ref, pden_ref, r_ref, out_ref):
    p = part_ref[0] + part_ref[1]
    d = pden_ref[0] + pden_ref[1]
    db = jnp.dot(d, r_ref[...], preferred_element_type=jnp.float32)
    out_ref[...] = p / jnp.where(db > 0.0, db, 1.0)


def _sc_body(npad, n_edges, src_hbm, dst_hbm, elp_hbm, erp_hbm, h_hbm,
             zacc_hbm, zden_hbm, part_hbm, pden_hbm, hrows, bufl, bufr, w_v,
             src_v, dst_v, sem, acc_sh, den_sh):
    c = lax.axis_index("c")
    s = lax.axis_index("s")
    wid = s * NC + c
    rows_per_tile = npad // NS

    row0 = s * rows_per_tile
    sizes = []
    left = rows_per_tile
    while left > 0:
        sz = min(left, CHUNK)
        sizes.append((rows_per_tile - left, sz))
        left -= sz
    pltpu.sync_copy(zacc_hbm.at[pl.ds(0, CHUNK)], hrows)
    pltpu.sync_copy(zden_hbm.at[pl.ds(0, CHUNK)], w_v)
    for off, sz in sizes:
        pltpu.sync_copy(hrows.at[pl.ds(0, sz)],
                        acc_sh.at[pl.ds(row0 + off, sz)])
        pltpu.sync_copy(w_v.at[pl.ds(0, sz)],
                        den_sh.at[pl.ds(row0 + off, sz)])
    plsc.subcore_barrier()

    n_chunks = n_edges // CHUNK
    n_workers = NC * NS
    iters = (n_chunks + n_workers - 1) // n_workers
    lane = lax.iota(jnp.int32, LN)

    @pl.loop(0, iters)
    def _chunk(i):
        cid = wid + i * n_workers

        @pl.when(cid < n_chunks)
        def _():
            off = cid * CHUNK
            pltpu.sync_copy(src_hbm.at[pl.ds(off, CHUNK)], src_v)
            pltpu.sync_copy(dst_hbm.at[pl.ds(off, CHUNK)], dst_v.at[0])
            ch = pltpu.async_copy(h_hbm.at[src_v], hrows, sem)
            cl = pltpu.async_copy(elp_hbm.at[src_v], bufl, sem)
            cr = pltpu.async_copy(erp_hbm.at[dst_v.at[0]], bufr, sem)
            ch.wait()
            cl.wait()
            cr.wait()

            @pl.loop(0, CHUNK, unroll=8)
            def _edge(e):
                t = bufl[e, :] + bufr[e, :]
                t = jnp.where(t > 0.0, t, 0.2 * t)
                w = jnp.where(lane < H, jnp.exp(t), 0.0)
                w_v[e, :] = w
                for head in range(H):
                    ws = w[head]
                    for half in range(F // LN):
                        col = head * F + half * LN
                        hrows[e, pl.ds(col, LN)] = (
                            hrows[e, pl.ds(col, LN)] * ws)

            pltpu.sync_copy(hrows, acc_sh.at[dst_v.at[0]], add=True)
            pltpu.sync_copy(w_v, den_sh.at[dst_v.at[0]], add=True)

    plsc.subcore_barrier()
    for off, sz in sizes:
        pltpu.sync_copy(acc_sh.at[pl.ds(row0 + off, sz)],
                        hrows.at[pl.ds(0, sz)])
        pltpu.sync_copy(hrows.at[pl.ds(0, sz)],
                        part_hbm.at[c, pl.ds(row0 + off, sz)])
        pltpu.sync_copy(den_sh.at[pl.ds(row0 + off, sz)],
                        w_v.at[pl.ds(0, sz)])
        pltpu.sync_copy(w_v.at[pl.ds(0, sz)],
                        pden_hbm.at[c, pl.ds(row0 + off, sz)])


def kernel(feat, edge_index, W, attn_l, attn_r):
    n, in_feats = feat.shape
    e = edge_index.shape[1]
    f32 = jnp.float32

    wt = W.T.astype(f32)
    eye = jnp.eye(H, dtype=f32)
    pad = jnp.zeros((HF, DENW - H), f32)
    a_l = (attn_l[0][:, :, None] * eye[:, None, :]).reshape(HF, H)
    a_r = (attn_r[0][:, :, None] * eye[:, None, :]).reshape(HF, H)
    a_l = jnp.concatenate([a_l, pad], axis=1)
    a_r = jnp.concatenate([a_r, pad], axis=1)
    src = edge_index[0].astype(jnp.int32)
    dst = edge_index[1].astype(jnp.int32)
    npad = -(-n // (NS * 8)) * (NS * 8)
    zacc = jnp.zeros((npad // NS, HF), f32)
    zden = jnp.zeros((npad // NS, DENW), f32)

    bn = 1000
    h, elp, erp = pl.pallas_call(
        _prep_body,
        grid=(n // bn,),
        in_specs=[
            pl.BlockSpec((bn, in_feats), lambda i: (i, 0)),
            pl.BlockSpec((in_feats, HF), lambda i: (0, 0)),
            pl.BlockSpec((HF, DENW), lambda i: (0, 0)),
            pl.BlockSpec((HF, DENW), lambda i: (0, 0)),
        ],
        out_specs=[
            pl.BlockSpec((bn, HF), lambda i: (i, 0)),
            pl.BlockSpec((bn, DENW), lambda i: (i, 0)),
            pl.BlockSpec((bn, DENW), lambda i: (i, 0)),
        ],
        out_shape=[
            jax.ShapeDtypeStruct((n, HF), f32),
            jax.ShapeDtypeStruct((n, DENW), f32),
            jax.ShapeDtypeStruct((n, DENW), f32),
        ],
    )(feat.astype(f32), wt, a_l, a_r)

    mesh = plsc.VectorSubcoreMesh(core_axis_name="c", subcore_axis_name="s",
                                  num_cores=NC, num_subcores=NS)
    sc = pl.kernel(
        functools.partial(_sc_body, npad, e),
        out_type=(
            jax.ShapeDtypeStruct((NC, npad, HF), f32),
            jax.ShapeDtypeStruct((NC, npad, DENW), f32),
        ),
        mesh=mesh,
        scratch_types=(
            pltpu.VMEM((CHUNK, HF), f32),
            pltpu.VMEM((CHUNK, DENW), f32),
            pltpu.VMEM((CHUNK, DENW), f32),
            pltpu.VMEM((CHUNK, DENW), f32),
            pltpu.VMEM((CHUNK,), jnp.int32),
            pltpu.VMEM((1, CHUNK), jnp.int32),
            pltpu.SemaphoreType.DMA,
            pltpu.VMEM_SHARED((npad, HF), f32),
            pltpu.VMEM_SHARED((npad, DENW), f32),
        ),
        compiler_params=pltpu.CompilerParams(
            needs_layout_passes=False, use_tc_tiling_on_sc=False),
    )
    part, pden = sc(src, dst, elp, erp, h, zacc, zden)

    r = jnp.concatenate(
        [jnp.kron(eye, jnp.ones((1, F), f32)),
         jnp.zeros((DENW - H, HF), f32)], axis=0)
    bo = npad // 8
    out = pl.pallas_call(
        _finish_body,
        grid=(npad // bo,),
        in_specs=[
            pl.BlockSpec((NC, bo, HF), lambda i: (0, i, 0)),
            pl.BlockSpec((NC, bo, DENW), lambda i: (0, i, 0)),
            pl.BlockSpec((DENW, HF), lambda i: (0, 0)),
        ],
        out_specs=pl.BlockSpec((bo, HF), lambda i: (i, 0)),
        out_shape=jax.ShapeDtypeStruct((npad, HF), f32),
    )(part, pden, r)
    return out[:n].reshape(n, H, F)


# speedup vs baseline: 1.6514x; 1.6514x over previous
import functools

import jax
import jax.numpy as jnp
from jax import lax
from jax.experimental import pallas as pl
from jax.experimental.pallas import tpu as pltpu
from jax.experimental.pallas import tpu_sc as plsc

NC = 2
NS = 16
LN = 16

H = 4
F = 32
HF = H * F
CHUNK = 128
DENW = 16


def _prep_body(feat_ref, wt_ref, al_ref, ar_ref, h_ref, elp_ref, erp_ref):
    hb = jnp.dot(feat_ref[...], wt_ref[...], preferred_element_type=jnp.float32)
    h_ref[...] = hb
    elp_ref[...] = jnp.dot(hb, al_ref[...], preferred_element_type=jnp.float32)
    erp_ref[...] = jnp.dot(hb, ar_ref[...], preferred_element_type=jnp.float32)


def _finish_body(part_ref, pden_ref, r_ref, out_ref):
    p = part_ref[0] + part_ref[1]
    d = pden_ref[0] + pden_ref[1]
    db = jnp.dot(d, r_ref[...], preferred_element_type=jnp.float32)
    out_ref[...] = p / jnp.where(db > 0.0, db, 1.0)


def _sc_body(npad, n_edges, src_hbm, dst_hbm, elp_hbm, erp_hbm, h_hbm,
             zacc_hbm, zden_hbm, part_hbm, pden_hbm, hrows, bl0, bl1, br0,
             br1, w_v, src0, src1, dst0, dst1, sem, sem0, sem1, acc_sh,
             den_sh):
    srcs, dsts = (src0, src1), (dst0, dst1)
    bls, brs, sems = (bl0, bl1), (br0, br1), (sem0, sem1)
    c = lax.axis_index("c")
    s = lax.axis_index("s")
    wid = s * NC + c
    rows_per_tile = npad // NS

    row0 = s * rows_per_tile
    sizes = []
    left = rows_per_tile
    while left > 0:
        sz = min(left, CHUNK)
        sizes.append((rows_per_tile - left, sz))
        left -= sz
    pltpu.sync_copy(zacc_hbm.at[pl.ds(0, CHUNK)], hrows)
    pltpu.sync_copy(zden_hbm.at[pl.ds(0, CHUNK)], w_v)
    for off, sz in sizes:
        pltpu.sync_copy(hrows.at[pl.ds(0, sz)],
                        acc_sh.at[pl.ds(row0 + off, sz)])
        pltpu.sync_copy(w_v.at[pl.ds(0, sz)],
                        den_sh.at[pl.ds(row0 + off, sz)])
    plsc.subcore_barrier()

    n_chunks = n_edges // CHUNK
    n_workers = NC * NS
    iters = (n_chunks + n_workers - 1) // n_workers
    lane = lax.iota(jnp.int32, LN)

    def fire_small(cid, sb, db, bl, br, sm):
        off = cid * CHUNK
        pltpu.sync_copy(src_hbm.at[pl.ds(off, CHUNK)], sb)
        pltpu.sync_copy(dst_hbm.at[pl.ds(off, CHUNK)], db.at[0])
        pltpu.async_copy(elp_hbm.at[sb], bl, sm)
        pltpu.async_copy(erp_hbm.at[db.at[0]], br, sm)

    def process(cid, p):
        sb, db, bl, br, sm = srcs[p], dsts[p], bls[p], brs[p], sems[p]
        q = 1 - p
        pltpu.make_async_copy(elp_hbm.at[sb], bl, sm).wait()
        pltpu.make_async_copy(erp_hbm.at[db.at[0]], br, sm).wait()

        @pl.loop(0, CHUNK)
        def _wloop(e):
            t = bl[e, :] + br[e, :]
            t = jnp.where(t > 0.0, t, 0.2 * t)
            w_v[e, :] = jnp.where(lane < H, jnp.exp(t), 0.0)

        nxt = cid + n_workers

        @pl.when(nxt < n_chunks)
        def _():
            fire_small(nxt, srcs[q], dsts[q], bls[q], brs[q], sems[q])

        pltpu.make_async_copy(h_hbm.at[sb], hrows, sem).wait()

        @pl.loop(0, CHUNK)
        def _mloop(e):
            wrow = w_v[e, :]
            for head in range(H):
                ws = wrow[head]
                for half in range(F // LN):
                    col = head * F + half * LN
                    hrows[e, pl.ds(col, LN)] = hrows[e, pl.ds(col, LN)] * ws

        pltpu.sync_copy(hrows, acc_sh.at[db.at[0]], add=True)
        pltpu.sync_copy(w_v, den_sh.at[db.at[0]], add=True)

        @pl.when(nxt < n_chunks)
        def _():
            pltpu.async_copy(h_hbm.at[srcs[q]], hrows, sem)

    fire_small(wid, srcs[0], dsts[0], bls[0], brs[0], sems[0])
    pltpu.async_copy(h_hbm.at[srcs[0]], hrows, sem)

    @pl.loop(0, (iters + 1) // 2)
    def _pair(j):
        for p in range(2):
            cid = wid + (2 * j + p) * n_workers

            @pl.when(cid < n_chunks)
            def _(cid=cid, p=p):
                process(cid, p)

    plsc.subcore_barrier()
    for off, sz in sizes:
        pltpu.sync_copy(acc_sh.at[pl.ds(row0 + off, sz)],
                        hrows.at[pl.ds(0, sz)])
        pltpu.sync_copy(hrows.at[pl.ds(0, sz)],
                        part_hbm.at[c, pl.ds(row0 + off, sz)])
        pltpu.sync_copy(den_sh.at[pl.ds(row0 + off, sz)],
                        w_v.at[pl.ds(0, sz)])
        pltpu.sync_copy(w_v.at[pl.ds(0, sz)],
                        pden_hbm.at[c, pl.ds(row0 + off, sz)])


def kernel(feat, edge_index, W, attn_l, attn_r):
    n, in_feats = feat.shape
    e = edge_index.shape[1]
    f32 = jnp.float32

    wt = W.T.astype(f32)
    eye = jnp.eye(H, dtype=f32)
    pad = jnp.zeros((HF, DENW - H), f32)
    a_l = (attn_l[0][:, :, None] * eye[:, None, :]).reshape(HF, H)
    a_r = (attn_r[0][:, :, None] * eye[:, None, :]).reshape(HF, H)
    a_l = jnp.concatenate([a_l, pad], axis=1)
    a_r = jnp.concatenate([a_r, pad], axis=1)
    src = edge_index[0].astype(jnp.int32)
    dst = edge_index[1].astype(jnp.int32)
    npad = -(-n // (NS * 8)) * (NS * 8)
    zacc = jnp.zeros((npad // NS, HF), f32)
    zden = jnp.zeros((npad // NS, DENW), f32)

    bn = 1000
    h, elp, erp = pl.pallas_call(
        _prep_body,
        grid=(n // bn,),
        in_specs=[
            pl.BlockSpec((bn, in_feats), lambda i: (i, 0)),
            pl.BlockSpec((in_feats, HF), lambda i: (0, 0)),
            pl.BlockSpec((HF, DENW), lambda i: (0, 0)),
            pl.BlockSpec((HF, DENW), lambda i: (0, 0)),
        ],
        out_specs=[
            pl.BlockSpec((bn, HF), lambda i: (i, 0)),
            pl.BlockSpec((bn, DENW), lambda i: (i, 0)),
            pl.BlockSpec((bn, DENW), lambda i: (i, 0)),
        ],
        out_shape=[
            jax.ShapeDtypeStruct((n, HF), f32),
            jax.ShapeDtypeStruct((n, DENW), f32),
            jax.ShapeDtypeStruct((n, DENW), f32),
        ],
    )(feat.astype(f32), wt, a_l, a_r)

    mesh = plsc.VectorSubcoreMesh(core_axis_name="c", subcore_axis_name="s",
                                  num_cores=NC, num_subcores=NS)
    sc = pl.kernel(
        functools.partial(_sc_body, npad, e),
        out_type=(
            jax.ShapeDtypeStruct((NC, npad, HF), f32),
            jax.ShapeDtypeStruct((NC, npad, DENW), f32),
        ),
        mesh=mesh,
        scratch_types=(
            pltpu.VMEM((CHUNK, HF), f32),
            pltpu.VMEM((CHUNK, DENW), f32),
            pltpu.VMEM((CHUNK, DENW), f32),
            pltpu.VMEM((CHUNK, DENW), f32),
            pltpu.VMEM((CHUNK, DENW), f32),
            pltpu.VMEM((CHUNK, DENW), f32),
            pltpu.VMEM((CHUNK,), jnp.int32),
            pltpu.VMEM((CHUNK,), jnp.int32),
            pltpu.VMEM((1, CHUNK), jnp.int32),
            pltpu.VMEM((1, CHUNK), jnp.int32),
            pltpu.SemaphoreType.DMA,
            pltpu.SemaphoreType.DMA,
            pltpu.SemaphoreType.DMA,
            pltpu.VMEM_SHARED((npad, HF), f32),
            pltpu.VMEM_SHARED((npad, DENW), f32),
        ),
        compiler_params=pltpu.CompilerParams(
            needs_layout_passes=False, use_tc_tiling_on_sc=False),
    )
    part, pden = sc(src, dst, elp, erp, h, zacc, zden)

    r = jnp.concatenate(
        [jnp.kron(eye, jnp.ones((1, F), f32)),
         jnp.zeros((DENW - H, HF), f32)], axis=0)
    bo = npad // 8
    out = pl.pallas_call(
        _finish_body,
        grid=(npad // bo,),
        in_specs=[
            pl.BlockSpec((NC, bo, HF), lambda i: (0, i, 0)),
            pl.BlockSpec((NC, bo, DENW), lambda i: (0, i, 0)),
            pl.BlockSpec((DENW, HF), lambda i: (0, 0)),
        ],
        out_specs=pl.BlockSpec((bo, HF), lambda i: (i, 0)),
        out_shape=jax.ShapeDtypeStruct((npad, HF), f32),
    )(part, pden, r)
    return out[:n].reshape(n, H, F)


# speedup vs baseline: 1.6774x; 1.0157x over previous
import functools

import jax
import jax.numpy as jnp
from jax import lax
from jax.experimental import pallas as pl
from jax.experimental.pallas import tpu as pltpu
from jax.experimental.pallas import tpu_sc as plsc

NC = 2
NS = 16
LN = 16

H = 4
F = 32
HF = H * F
CHUNK = 128
DENW = 16


def _prep_body(feat_ref, wt_ref, al_ref, ar_ref, h_ref, elp_ref, erp_ref):
    hb = jnp.dot(feat_ref[...], wt_ref[...], preferred_element_type=jnp.float32)
    h_ref[...] = hb
    elp_ref[...] = jnp.dot(hb, al_ref[...], preferred_element_type=jnp.float32)
    erp_ref[...] = jnp.dot(hb, ar_ref[...], preferred_element_type=jnp.float32)


def _finish_body(part_ref, pden_ref, r_ref, out_ref):
    p = part_ref[0] + part_ref[1]
    d = pden_ref[0] + pden_ref[1]
    db = jnp.dot(d, r_ref[...], preferred_element_type=jnp.float32)
    out_ref[...] = p / jnp.where(db > 0.0, db, 1.0)


def _sc_body(npad, n_edges, src_hbm, dst_hbm, elp_hbm, erp_hbm, h_hbm,
             zacc_hbm, zden_hbm, part_hbm, pden_hbm, hrows, bl0, bl1, br0,
             br1, w_v, src0, src1, dst0, dst1, sem, sem0, sem1, acc_sh,
             den_sh):
    srcs, dsts = (src0, src1), (dst0, dst1)
    bls, brs, sems = (bl0, bl1), (br0, br1), (sem0, sem1)
    c = lax.axis_index("c")
    s = lax.axis_index("s")
    wid = s * NC + c
    rows_per_tile = npad // NS

    row0 = s * rows_per_tile
    sizes = []
    left = rows_per_tile
    while left > 0:
        sz = min(left, CHUNK)
        sizes.append((rows_per_tile - left, sz))
        left -= sz
    pltpu.sync_copy(zacc_hbm.at[pl.ds(0, CHUNK)], hrows)
    pltpu.sync_copy(zden_hbm.at[pl.ds(0, CHUNK)], w_v)
    for off, sz in sizes:
        pltpu.sync_copy(hrows.at[pl.ds(0, sz)],
                        acc_sh.at[pl.ds(row0 + off, sz)])
        pltpu.sync_copy(w_v.at[pl.ds(0, sz)],
                        den_sh.at[pl.ds(row0 + off, sz)])
    plsc.subcore_barrier()

    n_chunks = n_edges // CHUNK
    n_workers = NC * NS
    iters = (n_chunks + n_workers - 1) // n_workers
    lane = lax.iota(jnp.int32, LN)

    def fire_small(cid, sb, db, bl, br, sm):
        off = cid * CHUNK
        pltpu.sync_copy(src_hbm.at[pl.ds(off, CHUNK)], sb)
        pltpu.sync_copy(dst_hbm.at[pl.ds(off, CHUNK)], db.at[0])
        pltpu.async_copy(elp_hbm.at[sb], bl, sm)
        pltpu.async_copy(erp_hbm.at[db.at[0]], br, sm)

    def process(cid, p):
        sb, db, bl, br, sm = srcs[p], dsts[p], bls[p], brs[p], sems[p]
        q = 1 - p
        pltpu.make_async_copy(elp_hbm.at[sb], bl, sm).wait()
        pltpu.make_async_copy(erp_hbm.at[db.at[0]], br, sm).wait()

        @pl.loop(0, CHUNK)
        def _wloop(e):
            t = bl[e, :] + br[e, :]
            t = jnp.where(t > 0.0, t, 0.2 * t)
            w_v[e, :] = jnp.where(lane < H, jnp.exp(t), 0.0)

        nxt = cid + n_workers

        @pl.when(nxt < n_chunks)
        def _():
            fire_small(nxt, srcs[q], dsts[q], bls[q], brs[q], sems[q])

        pltpu.make_async_copy(h_hbm.at[sb], hrows, sem).wait()

        @pl.loop(0, CHUNK, unroll=2)
        def _mloop(e):
            wrow = w_v[e, :]
            for head in range(H):
                ws = wrow[head]
                for half in range(F // LN):
                    col = head * F + half * LN
                    hrows[e, pl.ds(col, LN)] = hrows[e, pl.ds(col, LN)] * ws

        pltpu.sync_copy(hrows, acc_sh.at[db.at[0]], add=True)
        pltpu.sync_copy(w_v, den_sh.at[db.at[0]], add=True)

        @pl.when(nxt < n_chunks)
        def _():
            pltpu.async_copy(h_hbm.at[srcs[q]], hrows, sem)

    fire_small(wid, srcs[0], dsts[0], bls[0], brs[0], sems[0])
    pltpu.async_copy(h_hbm.at[srcs[0]], hrows, sem)

    @pl.loop(0, (iters + 1) // 2)
    def _pair(j):
        for p in range(2):
            cid = wid + (2 * j + p) * n_workers

            @pl.when(cid < n_chunks)
            def _(cid=cid, p=p):
                process(cid, p)

    plsc.subcore_barrier()
    for off, sz in sizes:
        pltpu.sync_copy(acc_sh.at[pl.ds(row0 + off, sz)],
                        hrows.at[pl.ds(0, sz)])
        pltpu.sync_copy(hrows.at[pl.ds(0, sz)],
                        part_hbm.at[c, pl.ds(row0 + off, sz)])
        pltpu.sync_copy(den_sh.at[pl.ds(row0 + off, sz)],
                        w_v.at[pl.ds(0, sz)])
        pltpu.sync_copy(w_v.at[pl.ds(0, sz)],
                        pden_hbm.at[c, pl.ds(row0 + off, sz)])


def kernel(feat, edge_index, W, attn_l, attn_r):
    n, in_feats = feat.shape
    e = edge_index.shape[1]
    f32 = jnp.float32

    wt = W.T.astype(f32)
    eye = jnp.eye(H, dtype=f32)
    pad = jnp.zeros((HF, DENW - H), f32)
    a_l = (attn_l[0][:, :, None] * eye[:, None, :]).reshape(HF, H)
    a_r = (attn_r[0][:, :, None] * eye[:, None, :]).reshape(HF, H)
    a_l = jnp.concatenate([a_l, pad], axis=1)
    a_r = jnp.concatenate([a_r, pad], axis=1)
    src = edge_index[0].astype(jnp.int32)
    dst = edge_index[1].astype(jnp.int32)
    npad = -(-n // (NS * 8)) * (NS * 8)
    zacc = jnp.zeros((npad // NS, HF), f32)
    zden = jnp.zeros((npad // NS, DENW), f32)

    bn = 1000
    h, elp, erp = pl.pallas_call(
        _prep_body,
        grid=(n // bn,),
        in_specs=[
            pl.BlockSpec((bn, in_feats), lambda i: (i, 0)),
            pl.BlockSpec((in_feats, HF), lambda i: (0, 0)),
            pl.BlockSpec((HF, DENW), lambda i: (0, 0)),
            pl.BlockSpec((HF, DENW), lambda i: (0, 0)),
        ],
        out_specs=[
            pl.BlockSpec((bn, HF), lambda i: (i, 0)),
            pl.BlockSpec((bn, DENW), lambda i: (i, 0)),
            pl.BlockSpec((bn, DENW), lambda i: (i, 0)),
        ],
        out_shape=[
            jax.ShapeDtypeStruct((n, HF), f32),
            jax.ShapeDtypeStruct((n, DENW), f32),
            jax.ShapeDtypeStruct((n, DENW), f32),
        ],
    )(feat.astype(f32), wt, a_l, a_r)

    mesh = plsc.VectorSubcoreMesh(core_axis_name="c", subcore_axis_name="s",
                                  num_cores=NC, num_subcores=NS)
    sc = pl.kernel(
        functools.partial(_sc_body, npad, e),
        out_type=(
            jax.ShapeDtypeStruct((NC, npad, HF), f32),
            jax.ShapeDtypeStruct((NC, npad, DENW), f32),
        ),
        mesh=mesh,
        scratch_types=(
            pltpu.VMEM((CHUNK, HF), f32),
            pltpu.VMEM((CHUNK, DENW), f32),
            pltpu.VMEM((CHUNK, DENW), f32),
            pltpu.VMEM((CHUNK, DENW), f32),
            pltpu.VMEM((CHUNK, DENW), f32),
            pltpu.VMEM((CHUNK, DENW), f32),
            pltpu.VMEM((CHUNK,), jnp.int32),
            pltpu.VMEM((CHUNK,), jnp.int32),
            pltpu.VMEM((1, CHUNK), jnp.int32),
            pltpu.VMEM((1, CHUNK), jnp.int32),
            pltpu.SemaphoreType.DMA,
            pltpu.SemaphoreType.DMA,
            pltpu.SemaphoreType.DMA,
            pltpu.VMEM_SHARED((npad, HF), f32),
            pltpu.VMEM_SHARED((npad, DENW), f32),
        ),
        compiler_params=pltpu.CompilerParams(
            needs_layout_passes=False, use_tc_tiling_on_sc=False),
    )
    part, pden = sc(src, dst, elp, erp, h, zacc, zden)

    r = jnp.concatenate(
        [jnp.kron(eye, jnp.ones((1, F), f32)),
         jnp.zeros((DENW - H, HF), f32)], axis=0)
    bo = npad // 8
    out = pl.pallas_call(
        _finish_body,
        grid=(npad // bo,),
        in_specs=[
            pl.BlockSpec((NC, bo, HF), lambda i: (0, i, 0)),
            pl.BlockSpec((NC, bo, DENW), lambda i: (0, i, 0)),
            pl.BlockSpec((DENW, HF), lambda i: (0, 0)),
        ],
        out_specs=pl.BlockSpec((bo, HF), lambda i: (i, 0)),
        out_shape=jax.ShapeDtypeStruct((npad, HF), f32),
    )(part, pden, r)
    return out[:n].reshape(n, H, F)


# speedup vs baseline: 1.9639x; 1.1708x over previous
import functools

import jax
import jax.numpy as jnp
from jax import lax
from jax.experimental import pallas as pl
from jax.experimental.pallas import tpu as pltpu
from jax.experimental.pallas import tpu_sc as plsc

NC = 2
NS = 16
LN = 16

H = 4
F = 32
HF = H * F
CHUNK = 128
DENW = 16


def _prep_body(feat_ref, wt_ref, al_ref, ar_ref, h_ref, elp_ref, erp_ref):
    hb = jnp.dot(feat_ref[...], wt_ref[...], preferred_element_type=jnp.float32)
    h_ref[...] = hb
    elp_ref[...] = jnp.dot(hb, al_ref[...], preferred_element_type=jnp.float32)
    erp_ref[...] = jnp.dot(hb, ar_ref[...], preferred_element_type=jnp.float32)


def _finish_body(part_ref, pden_ref, r_ref, out_ref):
    p = part_ref[0] + part_ref[1]
    d = pden_ref[0] + pden_ref[1]
    db = jnp.dot(d, r_ref[...], preferred_element_type=jnp.float32)
    out_ref[...] = p / jnp.where(db > 0.0, db, 1.0)


def _sc_body(npad, n_edges, src_hbm, dst_hbm, elp_hbm, erp_hbm, h_hbm,
             zacc_hbm, zden_hbm, part_hbm, pden_hbm, hrows, bl0, bl1, br0,
             br1, w_v, src0, src1, dst0, dst1, sem, sem0, sem1, acc_sh,
             den_sh):
    srcs, dsts = (src0, src1), (dst0, dst1)
    bls, brs, sems = (bl0, bl1), (br0, br1), (sem0, sem1)
    c = lax.axis_index("c")
    s = lax.axis_index("s")
    wid = s * NC + c
    rows_per_tile = npad // NS

    row0 = s * rows_per_tile
    sizes = []
    left = rows_per_tile
    while left > 0:
        sz = min(left, CHUNK)
        sizes.append((rows_per_tile - left, sz))
        left -= sz
    pltpu.sync_copy(zacc_hbm.at[pl.ds(0, CHUNK)], hrows)
    pltpu.sync_copy(zden_hbm.at[pl.ds(0, CHUNK)], w_v)
    for off, sz in sizes:
        pltpu.sync_copy(hrows.at[pl.ds(0, sz)],
                        acc_sh.at[pl.ds(row0 + off, sz)])
        pltpu.sync_copy(w_v.at[pl.ds(0, sz)],
                        den_sh.at[pl.ds(row0 + off, sz)])
    plsc.subcore_barrier()

    n_chunks = n_edges // CHUNK
    n_workers = NC * NS
    iters = (n_chunks + n_workers - 1) // n_workers
    lane = lax.iota(jnp.int32, LN)

    def fire_small(cid, sb, db, bl, br, sm):
        off = cid * CHUNK
        pltpu.sync_copy(src_hbm.at[pl.ds(off, CHUNK)], sb)
        pltpu.sync_copy(dst_hbm.at[pl.ds(off, CHUNK)], db.at[0])
        pltpu.async_copy(elp_hbm.at[sb], bl, sm)
        pltpu.async_copy(erp_hbm.at[db.at[0]], br, sm)

    def process(cid, p):
        sb, db, bl, br, sm = srcs[p], dsts[p], bls[p], brs[p], sems[p]
        q = 1 - p
        pltpu.make_async_copy(elp_hbm.at[sb], bl, sm).wait()
        pltpu.make_async_copy(erp_hbm.at[db.at[0]], br, sm).wait()

        @plsc.parallel_loop(0, CHUNK, unroll=2)
        def _wloop(e):
            t = bl[e, :] + br[e, :]
            t = jnp.where(t > 0.0, t, 0.2 * t)
            w_v[e, :] = jnp.where(lane < H, jnp.exp(t), 0.0)

        nxt = cid + n_workers

        @pl.when(nxt < n_chunks)
        def _():
            fire_small(nxt, srcs[q], dsts[q], bls[q], brs[q], sems[q])

        pltpu.make_async_copy(h_hbm.at[sb], hrows, sem).wait()

        @plsc.parallel_loop(0, CHUNK, unroll=2)
        def _mloop(e):
            wrow = w_v[e, :]
            for head in range(H):
                ws = wrow[head]
                for half in range(F // LN):
                    col = head * F + half * LN
                    hrows[e, pl.ds(col, LN)] = hrows[e, pl.ds(col, LN)] * ws

        pltpu.sync_copy(hrows, acc_sh.at[db.at[0]], add=True)
        pltpu.sync_copy(w_v, den_sh.at[db.at[0]], add=True)

        @pl.when(nxt < n_chunks)
        def _():
            pltpu.async_copy(h_hbm.at[srcs[q]], hrows, sem)

    fire_small(wid, srcs[0], dsts[0], bls[0], brs[0], sems[0])
    pltpu.async_copy(h_hbm.at[srcs[0]], hrows, sem)

    @pl.loop(0, (iters + 1) // 2)
    def _pair(j):
        for p in range(2):
            cid = wid + (2 * j + p) * n_workers

            @pl.when(cid < n_chunks)
            def _(cid=cid, p=p):
                process(cid, p)

    plsc.subcore_barrier()
    for off, sz in sizes:
        pltpu.sync_copy(acc_sh.at[pl.ds(row0 + off, sz)],
                        hrows.at[pl.ds(0, sz)])
        pltpu.sync_copy(hrows.at[pl.ds(0, sz)],
                        part_hbm.at[c, pl.ds(row0 + off, sz)])
        pltpu.sync_copy(den_sh.at[pl.ds(row0 + off, sz)],
                        w_v.at[pl.ds(0, sz)])
        pltpu.sync_copy(w_v.at[pl.ds(0, sz)],
                        pden_hbm.at[c, pl.ds(row0 + off, sz)])


def kernel(feat, edge_index, W, attn_l, attn_r):
    n, in_feats = feat.shape
    e = edge_index.shape[1]
    f32 = jnp.float32

    wt = W.T.astype(f32)
    eye = jnp.eye(H, dtype=f32)
    pad = jnp.zeros((HF, DENW - H), f32)
    a_l = (attn_l[0][:, :, None] * eye[:, None, :]).reshape(HF, H)
    a_r = (attn_r[0][:, :, None] * eye[:, None, :]).reshape(HF, H)
    a_l = jnp.concatenate([a_l, pad], axis=1)
    a_r = jnp.concatenate([a_r, pad], axis=1)
    src = edge_index[0].astype(jnp.int32)
    dst = edge_index[1].astype(jnp.int32)
    npad = -(-n // (NS * 8)) * (NS * 8)
    zacc = jnp.zeros((npad // NS, HF), f32)
    zden = jnp.zeros((npad // NS, DENW), f32)

    bn = 1000
    h, elp, erp = pl.pallas_call(
        _prep_body,
        grid=(n // bn,),
        in_specs=[
            pl.BlockSpec((bn, in_feats), lambda i: (i, 0)),
            pl.BlockSpec((in_feats, HF), lambda i: (0, 0)),
            pl.BlockSpec((HF, DENW), lambda i: (0, 0)),
            pl.BlockSpec((HF, DENW), lambda i: (0, 0)),
        ],
        out_specs=[
            pl.BlockSpec((bn, HF), lambda i: (i, 0)),
            pl.BlockSpec((bn, DENW), lambda i: (i, 0)),
            pl.BlockSpec((bn, DENW), lambda i: (i, 0)),
        ],
        out_shape=[
            jax.ShapeDtypeStruct((n, HF), f32),
            jax.ShapeDtypeStruct((n, DENW), f32),
            jax.ShapeDtypeStruct((n, DENW), f32),
        ],
    )(feat.astype(f32), wt, a_l, a_r)

    mesh = plsc.VectorSubcoreMesh(core_axis_name="c", subcore_axis_name="s",
                                  num_cores=NC, num_subcores=NS)
    sc = pl.kernel(
        functools.partial(_sc_body, npad, e),
        out_type=(
            jax.ShapeDtypeStruct((NC, npad, HF), f32),
            jax.ShapeDtypeStruct((NC, npad, DENW), f32),
        ),
        mesh=mesh,
        scratch_types=(
            pltpu.VMEM((CHUNK, HF), f32),
            pltpu.VMEM((CHUNK, DENW), f32),
            pltpu.VMEM((CHUNK, DENW), f32),
            pltpu.VMEM((CHUNK, DENW), f32),
            pltpu.VMEM((CHUNK, DENW), f32),
            pltpu.VMEM((CHUNK, DENW), f32),
            pltpu.VMEM((CHUNK,), jnp.int32),
            pltpu.VMEM((CHUNK,), jnp.int32),
            pltpu.VMEM((1, CHUNK), jnp.int32),
            pltpu.VMEM((1, CHUNK), jnp.int32),
            pltpu.SemaphoreType.DMA,
            pltpu.SemaphoreType.DMA,
            pltpu.SemaphoreType.DMA,
            pltpu.VMEM_SHARED((npad, HF), f32),
            pltpu.VMEM_SHARED((npad, DENW), f32),
        ),
        compiler_params=pltpu.CompilerParams(
            needs_layout_passes=False, use_tc_tiling_on_sc=False),
    )
    part, pden = sc(src, dst, elp, erp, h, zacc, zden)

    r = jnp.concatenate(
        [jnp.kron(eye, jnp.ones((1, F), f32)),
         jnp.zeros((DENW - H, HF), f32)], axis=0)
    bo = npad // 8
    out = pl.pallas_call(
        _finish_body,
        grid=(npad // bo,),
        in_specs=[
            pl.BlockSpec((NC, bo, HF), lambda i: (0, i, 0)),
            pl.BlockSpec((NC, bo, DENW), lambda i: (0, i, 0)),
            pl.BlockSpec((DENW, HF), lambda i: (0, 0)),
        ],
        out_specs=pl.BlockSpec((bo, HF), lambda i: (i, 0)),
        out_shape=jax.ShapeDtypeStruct((npad, HF), f32),
    )(part, pden, r)
    return out[:n].reshape(n, H, F)
